# baseline (speedup 1.0000x reference)
import functools

import jax
import jax.numpy as jnp
from jax import lax
from jax.experimental import pallas as pl
from jax.experimental.pallas import tpu as pltpu
from jax.experimental.pallas import tpu_sc as plsc

T, D, E, H = 8192, 768, 16, 128
BA = 256
BC = 256
G = 48

_NW = 32
_NC = 2
TPW = T // _NW
CH = 64
DCH = 128


def _gate_body(x_ref, wg_ref, bg_ref, ones_ref, eidx_ref, bcnt_ref):
    xb = x_ref[...]
    logits = lax.dot_general(
        xb, wg_ref[...], (((1,), (1,)), ((), ())),
        preferred_element_type=jnp.float32) + bg_ref[...]
    eidx = jnp.argmax(logits, axis=1)
    lane = lax.broadcasted_iota(jnp.int32, (BA, E), 1)
    oh = (eidx[:, None] == lane).astype(jnp.bfloat16)
    bcnt = lax.dot_general(ones_ref[...], oh,
                           (((1,), (0,)), ((), ())),
                           preferred_element_type=jnp.float32)
    eidx_ref[...] = eidx
    bcnt_ref[...] = jnp.round(bcnt).astype(jnp.int32)[None]


def _gate_call(x, Wg, bg, Tl=T):
    nblk = Tl // BA
    ones = jnp.ones((1, BA), jnp.bfloat16)
    return pl.pallas_call(
        _gate_body,
        grid=(nblk,),
        in_specs=[
            pl.BlockSpec((BA, D), lambda i: (i, 0)),
            pl.BlockSpec((E, D), lambda i: (0, 0)),
            pl.BlockSpec((1, E), lambda i: (0, 0)),
            pl.BlockSpec((1, BA), lambda i: (0, 0)),
        ],
        out_specs=[
            pl.BlockSpec((BA,), lambda i: (i,)),
            pl.BlockSpec((1, 1, E), lambda i: (i, 0, 0)),
        ],
        out_shape=[
            jax.ShapeDtypeStruct((Tl,), jnp.int32),
            jax.ShapeDtypeStruct((nblk, 1, E), jnp.int32),
        ],
    )(x, Wg, bg.reshape(1, E), ones)


def _scatter_body(tpw, gl, nbk, x_hbm, eidx_hbm, bcnt_hbm,
                  xs_hbm, dest_hbm, be_hbm, bt_hbm, lo_hbm, hi_hbm, fi_hbm,
                  xbuf, xbuf2, ebuf, ebuf2, dbuf, dbuf2, cbuf,
                  bebuf, btbuf, lobuf, hibuf, fibuf,
                  isem0, isem1, osem0, osem1):
    wid = lax.axis_index("s") * _NC + lax.axis_index("c")
    base = wid * tpw
    pltpu.sync_copy(bcnt_hbm, cbuf)
    pt = jnp.zeros((16,), jnp.int32)
    tot = jnp.zeros((16,), jnp.int32)
    for w in range(nbk):
        bc_w = cbuf[pl.ds(w * E, E)]
        tot = tot + bc_w
        coef = jnp.where(w < wid, jnp.int32(1), jnp.int32(0))
        pt = pt + bc_w * coef
    cv = tot

    starts = []
    ends = []
    acc = jnp.int32(0)
    for e in range(E):
        starts.append(acc)
        acc = acc + cv[e]
        ends.append(acc)

    lane16 = lax.iota(jnp.int32, 16)
    run_vec = pt
    for e in range(E):
        run_vec = run_vec + jnp.where(lane16 == e, starts[e], 0)

    zero16 = jnp.zeros((16,), jnp.int32)

    nch = tpw // CH
    xbufs = (xbuf, xbuf2)
    ebufs = (ebuf, ebuf2)
    dbufs = (dbuf, dbuf2)
    isems = (isem0, isem1)
    osems = (osem0, osem1)
    ins = [None] * nch
    outs = [None] * nch

    def start_in(ci):
        b = ci % 2
        off = base + ci * CH
        ins[ci] = (
            pltpu.async_copy(x_hbm.at[pl.ds(off, CH)], xbufs[b], isems[b]),
            pltpu.async_copy(eidx_hbm.at[pl.ds(off, CH)], ebufs[b], isems[b]),
        )

    start_in(0)
    for ci in range(nch):
        b = ci % 2
        if ci >= 1:
            outs[ci - 1].wait()
        if ci + 1 < nch:
            start_in(ci + 1)
        for cp in ins[ci]:
            cp.wait()
        for j in range(CH // 16):
            ev = ebufs[b][pl.ds(j * 16, 16)]
            dv = zero16
            hist = zero16
            for k in range(16):
                bk = zero16 + ev[k]
                same = jnp.where(ev == bk, 1, 0)
                after = jnp.where(lane16 > k, 1, 0)
                dv = dv + same * after
                hist = hist + jnp.where(lane16 == bk, 1, 0)
            for e in range(E):
                dv = dv + jnp.where(ev == e, run_vec[e], 0)
            run_vec = run_vec + hist
            dbufs[b][pl.ds(j * 16, 16)] = dv
        outs[ci] = pltpu.async_copy(xbufs[b], xs_hbm.at[dbufs[b]], osems[b])
        pltpu.sync_copy(dbufs[b], dest_hbm.at[pl.ds(base + ci * CH, CH)])
    outs[nch - 1].wait()

    @pl.when(wid == 0)
    def _():
        ft = []
        offs = []
        tpr = []
        acc2 = jnp.int32(0)
        for e in range(E):
            nonzero = cv[e] > 0
            ft_e = starts[e] // BC
            lt_e = jnp.where(nonzero, (ends[e] - 1) // BC, ft_e)
            tpr_e = jnp.where(nonzero, lt_e - ft_e + 1, 0)
            ft.append(ft_e)
            offs.append(acc2)
            tpr.append(tpr_e)
            acc2 = acc2 + tpr_e
        cum_t = [offs[e] + tpr[e] for e in range(E)]
        total = acc2
        tl = total - 1
        be_l = jnp.int32(0)
        for e in range(E):
            be_l = be_l + jnp.where(cum_t[e] <= tl, 1, 0)
        be_l = jnp.minimum(be_l, E - 1)
        bt_l = jnp.int32(0)
        for e in range(E):
            bt_l = bt_l + jnp.where(be_l == e, ft[e] + (tl - offs[e]), 0)

        def sched(gv, validv):
            bev = jnp.zeros((16,), jnp.int32)
            for e in range(E):
                bev = bev + jnp.where(cum_t[e] <= gv, 1, 0)
            bev = jnp.minimum(bev, E - 1)
            btv = jnp.zeros((16,), jnp.int32)
            lov = jnp.zeros((16,), jnp.int32)
            hiv = jnp.zeros((16,), jnp.int32)
            for e in range(E):
                sel = bev == e
                btv = btv + jnp.where(sel, ft[e] + (gv - offs[e]), 0)
                lov = lov + jnp.where(sel, starts[e], 0)
                hiv = hiv + jnp.where(sel, ends[e], 0)
            bev = jnp.where(validv, bev, be_l)
            btv = jnp.where(validv, btv, bt_l)
            lov = jnp.where(validv, lov, 0)
            hiv = jnp.where(validv, hiv, 0)
            return bev, btv, lov, hiv

        for v in range(gl // 16):
            gv = lax.iota(jnp.int32, 16) + 16 * v
            valid = gv < total
            bev, btv, lov, hiv = sched(gv, valid)
            gp = gv - 1
            _, btp, _, _ = sched(gp, (gp >= 0) & (gp < total))
            fiv = jnp.where(gv == 0, 1,
                            jnp.where(valid & (btv != btp), 1, 0))
            sl = pl.ds(v * 16, 16)
            bebuf[sl] = bev
            btbuf[sl] = btv
            lobuf[sl] = lov
            hibuf[sl] = hiv
            fibuf[sl] = fiv
        pltpu.sync_copy(bebuf, be_hbm)
        pltpu.sync_copy(btbuf, bt_hbm)
        pltpu.sync_copy(lobuf, lo_hbm)
        pltpu.sync_copy(hibuf, hi_hbm)
        pltpu.sync_copy(fibuf, fi_hbm)


@functools.cache
def _make_scatter_call(Tl=T, gl=G):
    meta = jax.ShapeDtypeStruct((gl,), jnp.int32)
    tpw = Tl // _NW
    nbk = Tl // BA
    return functools.partial(
        pl.kernel,
        mesh=plsc.VectorSubcoreMesh(core_axis_name="c", subcore_axis_name="s"),
        out_type=[
            jax.ShapeDtypeStruct((Tl, D), jnp.float32),
            jax.ShapeDtypeStruct((Tl,), jnp.int32),
            meta, meta, meta, meta, meta,
        ],
        scratch_types=[
            pltpu.VMEM((CH, D), jnp.float32),
            pltpu.VMEM((CH, D), jnp.float32),
            pltpu.VMEM((CH,), jnp.int32),
            pltpu.VMEM((CH,), jnp.int32),
            pltpu.VMEM((CH,), jnp.int32),
            pltpu.VMEM((CH,), jnp.int32),
            pltpu.VMEM((nbk * E,), jnp.int32),
            pltpu.VMEM((gl,), jnp.int32),
            pltpu.VMEM((gl,), jnp.int32),
            pltpu.VMEM((gl,), jnp.int32),
            pltpu.VMEM((gl,), jnp.int32),
            pltpu.VMEM((gl,), jnp.int32),
            pltpu.SemaphoreType.DMA,
            pltpu.SemaphoreType.DMA,
            pltpu.SemaphoreType.DMA,
            pltpu.SemaphoreType.DMA,
        ],
    )(functools.partial(_scatter_body, tpw, gl, nbk))


def _mlp_body(be_ref, bt_ref, lo_ref, hi_ref, fi_ref,
              xs_ref, w1_ref, b1_ref, w2_ref, b2_ref, w3_ref, b3_ref,
              out_ref):
    g = pl.program_id(0)
    e = be_ref[g]
    m = bt_ref[g]
    lo = lo_ref[g]
    hi = hi_ref[g]
    first = fi_ref[g]

    xb = xs_ref[...].astype(jnp.bfloat16)
    h1 = jax.nn.relu(lax.dot_general(
        xb, w1_ref[0].astype(jnp.bfloat16), (((1,), (1,)), ((), ())),
        preferred_element_type=jnp.float32) + b1_ref[0])
    h2 = jax.nn.relu(lax.dot_general(
        h1.astype(jnp.bfloat16), w2_ref[0].astype(jnp.bfloat16),
        (((1,), (1,)), ((), ())),
        preferred_element_type=jnp.float32) + b2_ref[0])
    lane = lax.broadcasted_iota(jnp.int32, (1, E), 1)
    b3v = jnp.sum(jnp.where(lane == e, b3_ref[...], 0.0))
    o = jnp.sum(h2 * w3_ref[0], axis=1) + b3v
    rows = m * BC + lax.broadcasted_iota(jnp.int32, (BC,), 0)
    mask = (rows >= lo) & (rows < hi)
    masked = jnp.where(mask, o, 0.0)
    prev = jnp.where(first == 1, jnp.zeros((BC,), jnp.float32), out_ref[...])
    out_ref[...] = prev + masked


def _mlp_call(meta, xs, W1, b1, W2, b2, W3r, b3r, Tl=T, gl=G):
    block_expert, block_tile, row_lo, row_hi, is_first = meta
    spec = pltpu.PrefetchScalarGridSpec(
        num_scalar_prefetch=5,
        grid=(gl,),
        in_specs=[
            pl.BlockSpec((BC, D), lambda g, be, bt, lo, hi, fi: (bt[g], 0)),
            pl.BlockSpec((1, H, D), lambda g, be, bt, lo, hi, fi: (be[g], 0, 0)),
            pl.BlockSpec((1, 1, H), lambda g, be, bt, lo, hi, fi: (be[g], 0, 0)),
            pl.BlockSpec((1, H, H), lambda g, be, bt, lo, hi, fi: (be[g], 0, 0)),
            pl.BlockSpec((1, 1, H), lambda g, be, bt, lo, hi, fi: (be[g], 0, 0)),
            pl.BlockSpec((1, 1, H), lambda g, be, bt, lo, hi, fi: (be[g], 0, 0)),
            pl.BlockSpec((1, E), lambda g, be, bt, lo, hi, fi: (0, 0)),
        ],
        out_specs=pl.BlockSpec((BC,), lambda g, be, bt, lo, hi, fi: (bt[g],)),
    )
    return pl.pallas_call(
        _mlp_body,
        grid_spec=spec,
        out_shape=jax.ShapeDtypeStruct((Tl,), jnp.float32),
    )(block_expert, block_tile, row_lo, row_hi, is_first,
      xs, W1, b1, W2, b2, W3r, b3r)


def _unpermute_body(tpw, dch, os_hbm, dest_hbm, out_hbm, dbuf, vbuf, sem):
    wid = lax.axis_index("s") * _NC + lax.axis_index("c")
    base = wid * tpw
    for ci in range(tpw // dch):
        off = base + ci * DCH
        pltpu.sync_copy(dest_hbm.at[pl.ds(off, dch)], dbuf)
        pltpu.async_copy(os_hbm.at[dbuf], vbuf, sem).wait()
        pltpu.sync_copy(vbuf, out_hbm.at[pl.ds(off, dch)])


@functools.cache
def _make_unpermute_call(Tl=T):
    tpw = Tl // _NW
    dch = min(DCH, tpw)
    return functools.partial(
        pl.kernel,
        mesh=plsc.VectorSubcoreMesh(core_axis_name="c", subcore_axis_name="s"),
        out_type=jax.ShapeDtypeStruct((Tl,), jnp.float32),
        scratch_types=[
            pltpu.VMEM((dch,), jnp.int32),
            pltpu.VMEM((dch,), jnp.float32),
            pltpu.SemaphoreType.DMA,
        ],
    )(functools.partial(_unpermute_body, tpw, dch))


def _half(xh, Wg, bg, W1, b1, W2, b2, W3, b3, Tl):
    gl = Tl // BC + E
    eidx, bcnt = _gate_call(xh, Wg, bg, Tl)
    xs, dest, be, bt, lo, hi, fi = _make_scatter_call(Tl, gl)(
        xh, eidx, bcnt.reshape(Tl // BA * E))
    out_sorted = _mlp_call(
        (be, bt, lo, hi, fi),
        xs, W1, b1.reshape(E, 1, H), W2, b2.reshape(E, 1, H),
        W3.reshape(E, 1, H), b3.reshape(1, E), Tl, gl)
    return _make_unpermute_call(Tl)(out_sorted, dest)


def kernel(x, Wg, bg, W1, b1, W2, b2, W3, b3):
    Tl = T // 2
    o0 = _half(x[:Tl], Wg, bg, W1, b1, W2, b2, W3, b3, Tl)
    o1 = _half(x[Tl:], Wg, bg, W1, b1, W2, b2, W3, b3, Tl)
    return jnp.concatenate([o0, o1])[:, None]


# speedup vs baseline: 1.1832x; 1.1832x over previous
import functools

import jax
import jax.numpy as jnp
from jax import lax
from jax.experimental import pallas as pl
from jax.experimental.pallas import tpu as pltpu
from jax.experimental.pallas import tpu_sc as plsc

T, D, E, H = 8192, 768, 16, 128
BA = 256
BC = 256
G = 48

_NW = 32
_NC = 2
TPW = T // _NW
CH = 64
DCH = 128


def _gate_body(x_ref, wg_ref, bg_ref, ones_ref, eidx_ref, bcnt_ref):
    xb = x_ref[...]
    logits = lax.dot_general(
        xb, wg_ref[...], (((1,), (1,)), ((), ())),
        preferred_element_type=jnp.float32) + bg_ref[...]
    eidx = jnp.argmax(logits, axis=1)
    lane = lax.broadcasted_iota(jnp.int32, (BA, E), 1)
    oh = (eidx[:, None] == lane).astype(jnp.bfloat16)
    bcnt = lax.dot_general(ones_ref[...], oh,
                           (((1,), (0,)), ((), ())),
                           preferred_element_type=jnp.float32)
    eidx_ref[...] = eidx
    bcnt_ref[...] = jnp.round(bcnt).astype(jnp.int32)[None]


def _gate_call(x, Wg, bg):
    nblk = T // BA
    ones = jnp.ones((1, BA), jnp.bfloat16)
    return pl.pallas_call(
        _gate_body,
        grid=(nblk,),
        in_specs=[
            pl.BlockSpec((BA, D), lambda i: (i, 0)),
            pl.BlockSpec((E, D), lambda i: (0, 0)),
            pl.BlockSpec((1, E), lambda i: (0, 0)),
            pl.BlockSpec((1, BA), lambda i: (0, 0)),
        ],
        out_specs=[
            pl.BlockSpec((BA,), lambda i: (i,)),
            pl.BlockSpec((1, 1, E), lambda i: (i, 0, 0)),
        ],
        out_shape=[
            jax.ShapeDtypeStruct((T,), jnp.int32),
            jax.ShapeDtypeStruct((nblk, 1, E), jnp.int32),
        ],
    )(x, Wg, bg.reshape(1, E), ones)


def _scatter_body(x_hbm, eidx_hbm, bcnt_hbm,
                  xs_hbm, dest_hbm, be_hbm, bt_hbm, lo_hbm, hi_hbm, fi_hbm,
                  xbuf, xbuf2, ebuf, ebuf2, dbuf, dbuf2, cbuf,
                  bebuf, btbuf, lobuf, hibuf, fibuf,
                  isem0, isem1, osem0, osem1):
    wid = lax.axis_index("s") * _NC + lax.axis_index("c")
    base = wid * TPW
    pltpu.sync_copy(bcnt_hbm, cbuf)
    pt = jnp.zeros((16,), jnp.int32)
    tot = jnp.zeros((16,), jnp.int32)
    for w in range(_NW):
        bc_w = cbuf[pl.ds(w * E, E)]
        tot = tot + bc_w
        coef = jnp.where(w < wid, jnp.int32(1), jnp.int32(0))
        pt = pt + bc_w * coef
    cv = tot

    starts = []
    ends = []
    acc = jnp.int32(0)
    for e in range(E):
        starts.append(acc)
        acc = acc + cv[e]
        ends.append(acc)

    lane16 = lax.iota(jnp.int32, 16)
    run_vec = pt
    for e in range(E):
        run_vec = run_vec + jnp.where(lane16 == e, starts[e], 0)

    zero16 = jnp.zeros((16,), jnp.int32)

    nch = TPW // CH
    xbufs = (xbuf, xbuf2)
    ebufs = (ebuf, ebuf2)
    dbufs = (dbuf, dbuf2)
    isems = (isem0, isem1)
    osems = (osem0, osem1)
    ins = [None] * nch
    outs = [None] * nch

    def start_in(ci):
        b = ci % 2
        off = base + ci * CH
        ins[ci] = (
            pltpu.async_copy(x_hbm.at[pl.ds(off, CH)], xbufs[b], isems[b]),
            pltpu.async_copy(eidx_hbm.at[pl.ds(off, CH)], ebufs[b], isems[b]),
        )

    start_in(0)
    for ci in range(nch):
        b = ci % 2
        if ci >= 1:
            outs[ci - 1].wait()
        if ci + 1 < nch:
            start_in(ci + 1)
        for cp in ins[ci]:
            cp.wait()
        for j in range(CH // 16):
            ev = ebufs[b][pl.ds(j * 16, 16)]
            dv = zero16
            hist = zero16
            for k in range(16):
                bk = zero16 + ev[k]
                same = jnp.where(ev == bk, 1, 0)
                after = jnp.where(lane16 > k, 1, 0)
                dv = dv + same * after
                hist = hist + jnp.where(lane16 == bk, 1, 0)
            for e in range(E):
                dv = dv + jnp.where(ev == e, run_vec[e], 0)
            run_vec = run_vec + hist
            dbufs[b][pl.ds(j * 16, 16)] = dv
        outs[ci] = pltpu.async_copy(xbufs[b], xs_hbm.at[dbufs[b]], osems[b])
        pltpu.sync_copy(dbufs[b], dest_hbm.at[pl.ds(base + ci * CH, CH)])
    outs[nch - 1].wait()

    @pl.when(wid == 0)
    def _():
        ft = []
        offs = []
        tpr = []
        acc2 = jnp.int32(0)
        for e in range(E):
            nonzero = cv[e] > 0
            ft_e = starts[e] // BC
            lt_e = jnp.where(nonzero, (ends[e] - 1) // BC, ft_e)
            tpr_e = jnp.where(nonzero, lt_e - ft_e + 1, 0)
            ft.append(ft_e)
            offs.append(acc2)
            tpr.append(tpr_e)
            acc2 = acc2 + tpr_e
        cum_t = [offs[e] + tpr[e] for e in range(E)]
        total = acc2
        tl = total - 1
        be_l = jnp.int32(0)
        for e in range(E):
            be_l = be_l + jnp.where(cum_t[e] <= tl, 1, 0)
        be_l = jnp.minimum(be_l, E - 1)
        bt_l = jnp.int32(0)
        for e in range(E):
            bt_l = bt_l + jnp.where(be_l == e, ft[e] + (tl - offs[e]), 0)

        def sched(gv, validv):
            bev = jnp.zeros((16,), jnp.int32)
            for e in range(E):
                bev = bev + jnp.where(cum_t[e] <= gv, 1, 0)
            bev = jnp.minimum(bev, E - 1)
            btv = jnp.zeros((16,), jnp.int32)
            lov = jnp.zeros((16,), jnp.int32)
            hiv = jnp.zeros((16,), jnp.int32)
            for e in range(E):
                sel = bev == e
                btv = btv + jnp.where(sel, ft[e] + (gv - offs[e]), 0)
                lov = lov + jnp.where(sel, starts[e], 0)
                hiv = hiv + jnp.where(sel, ends[e], 0)
            bev = jnp.where(validv, bev, be_l)
            btv = jnp.where(validv, btv, bt_l)
            lov = jnp.where(validv, lov, 0)
            hiv = jnp.where(validv, hiv, 0)
            return bev, btv, lov, hiv

        for v in range(G // 16):
            gv = lax.iota(jnp.int32, 16) + 16 * v
            valid = gv < total
            bev, btv, lov, hiv = sched(gv, valid)
            gp = gv - 1
            _, btp, _, _ = sched(gp, (gp >= 0) & (gp < total))
            fiv = jnp.where(gv == 0, 1,
                            jnp.where(valid & (btv != btp), 1, 0))
            sl = pl.ds(v * 16, 16)
            bebuf[sl] = bev
            btbuf[sl] = btv
            lobuf[sl] = lov
            hibuf[sl] = hiv
            fibuf[sl] = fiv
        pltpu.sync_copy(bebuf, be_hbm)
        pltpu.sync_copy(btbuf, bt_hbm)
        pltpu.sync_copy(lobuf, lo_hbm)
        pltpu.sync_copy(hibuf, hi_hbm)
        pltpu.sync_copy(fibuf, fi_hbm)


@functools.cache
def _make_scatter_call():
    meta = jax.ShapeDtypeStruct((G,), jnp.int32)
    return functools.partial(
        pl.kernel,
        mesh=plsc.VectorSubcoreMesh(core_axis_name="c", subcore_axis_name="s"),
        out_type=[
            jax.ShapeDtypeStruct((T, D), jnp.float32),
            jax.ShapeDtypeStruct((T,), jnp.int32),
            meta, meta, meta, meta, meta,
        ],
        scratch_types=[
            pltpu.VMEM((CH, D), jnp.float32),
            pltpu.VMEM((CH, D), jnp.float32),
            pltpu.VMEM((CH,), jnp.int32),
            pltpu.VMEM((CH,), jnp.int32),
            pltpu.VMEM((CH,), jnp.int32),
            pltpu.VMEM((CH,), jnp.int32),
            pltpu.VMEM((T // BA * E,), jnp.int32),
            pltpu.VMEM((G,), jnp.int32),
            pltpu.VMEM((G,), jnp.int32),
            pltpu.VMEM((G,), jnp.int32),
            pltpu.VMEM((G,), jnp.int32),
            pltpu.VMEM((G,), jnp.int32),
            pltpu.SemaphoreType.DMA,
            pltpu.SemaphoreType.DMA,
            pltpu.SemaphoreType.DMA,
            pltpu.SemaphoreType.DMA,
        ],
    )(_scatter_body)


def _mlp_body(be_ref, bt_ref, lo_ref, hi_ref, fi_ref,
              xs_ref, w1_ref, b1_ref, w2_ref, b2_ref, w3_ref, b3_ref,
              out_ref):
    g = pl.program_id(0)
    e = be_ref[g]
    m = bt_ref[g]
    lo = lo_ref[g]
    hi = hi_ref[g]
    first = fi_ref[g]

    xb = xs_ref[...].astype(jnp.bfloat16)
    h1 = jax.nn.relu(lax.dot_general(
        xb, w1_ref[0].astype(jnp.bfloat16), (((1,), (1,)), ((), ())),
        preferred_element_type=jnp.float32) + b1_ref[0])
    h2 = jax.nn.relu(lax.dot_general(
        h1.astype(jnp.bfloat16), w2_ref[0].astype(jnp.bfloat16),
        (((1,), (1,)), ((), ())),
        preferred_element_type=jnp.float32) + b2_ref[0])
    lane = lax.broadcasted_iota(jnp.int32, (1, E), 1)
    b3v = jnp.sum(jnp.where(lane == e, b3_ref[...], 0.0))
    o = jnp.sum(h2 * w3_ref[0], axis=1) + b3v
    rows = m * BC + lax.broadcasted_iota(jnp.int32, (BC,), 0)
    mask = (rows >= lo) & (rows < hi)
    masked = jnp.where(mask, o, 0.0)
    prev = jnp.where(first == 1, jnp.zeros((BC,), jnp.float32), out_ref[...])
    out_ref[...] = prev + masked


def _mlp_call(meta, xs, W1, b1, W2, b2, W3r, b3r):
    block_expert, block_tile, row_lo, row_hi, is_first = meta
    spec = pltpu.PrefetchScalarGridSpec(
        num_scalar_prefetch=5,
        grid=(G,),
        in_specs=[
            pl.BlockSpec((BC, D), lambda g, be, bt, lo, hi, fi: (bt[g], 0)),
            pl.BlockSpec((1, H, D), lambda g, be, bt, lo, hi, fi: (be[g], 0, 0)),
            pl.BlockSpec((1, 1, H), lambda g, be, bt, lo, hi, fi: (be[g], 0, 0)),
            pl.BlockSpec((1, H, H), lambda g, be, bt, lo, hi, fi: (be[g], 0, 0)),
            pl.BlockSpec((1, 1, H), lambda g, be, bt, lo, hi, fi: (be[g], 0, 0)),
            pl.BlockSpec((1, 1, H), lambda g, be, bt, lo, hi, fi: (be[g], 0, 0)),
            pl.BlockSpec((1, E), lambda g, be, bt, lo, hi, fi: (0, 0)),
        ],
        out_specs=pl.BlockSpec((BC,), lambda g, be, bt, lo, hi, fi: (bt[g],)),
    )
    return pl.pallas_call(
        _mlp_body,
        grid_spec=spec,
        out_shape=jax.ShapeDtypeStruct((T,), jnp.float32),
    )(block_expert, block_tile, row_lo, row_hi, is_first,
      xs, W1, b1, W2, b2, W3r, b3r)


def _unpermute_body(os_hbm, dest_hbm, out_hbm, dbuf, vbuf, sem):
    wid = lax.axis_index("s") * _NC + lax.axis_index("c")
    base = wid * TPW
    for ci in range(TPW // DCH):
        off = base + ci * DCH
        pltpu.sync_copy(dest_hbm.at[pl.ds(off, DCH)], dbuf)
        pltpu.async_copy(os_hbm.at[dbuf], vbuf, sem).wait()
        pltpu.sync_copy(vbuf, out_hbm.at[pl.ds(off, DCH)])


@functools.cache
def _make_unpermute_call():
    return functools.partial(
        pl.kernel,
        mesh=plsc.VectorSubcoreMesh(core_axis_name="c", subcore_axis_name="s"),
        out_type=jax.ShapeDtypeStruct((T,), jnp.float32),
        scratch_types=[
            pltpu.VMEM((DCH,), jnp.int32),
            pltpu.VMEM((DCH,), jnp.float32),
            pltpu.SemaphoreType.DMA,
        ],
    )(_unpermute_body)


def kernel(x, Wg, bg, W1, b1, W2, b2, W3, b3):
    eidx, bcnt = _gate_call(x, Wg, bg)
    xs, dest, be, bt, lo, hi, fi = _make_scatter_call()(
        x, eidx, bcnt.reshape(T // BA * E))
    out_sorted = _mlp_call(
        (be, bt, lo, hi, fi),
        xs, W1, b1.reshape(E, 1, H), W2, b2.reshape(E, 1, H),
        W3.reshape(E, 1, H), b3.reshape(1, E))
    out = _make_unpermute_call()(out_sorted, dest)
    return out[:, None]
